# baseline (speedup 1.0000x reference)
import functools
import math

import jax
import jax.numpy as jnp
from jax import lax
from jax.experimental import pallas as pl
from jax.experimental.pallas import tpu as pltpu
from jax.experimental.pallas import tpu_sc as plsc

D = 64
SCALE = math.sqrt(D)
B_TOKENS = 4096 * 200
NC = 2
NS = 16
NW = NC * NS
PER_W = B_TOKENS // NW
C = 128
NBUF = 4
NCHUNK = PER_W // C
NGROUP = NCHUNK // NBUF


def _build_sc_embed():
  mesh = plsc.VectorSubcoreMesh(core_axis_name="c", subcore_axis_name="s")

  @functools.partial(
      pl.kernel,
      mesh=mesh,
      out_type=jax.ShapeDtypeStruct((B_TOKENS, D), jnp.float32),
      compiler_params=pltpu.CompilerParams(use_tc_tiling_on_sc=False),
      scratch_types=[
          pltpu.VMEM((NBUF, C), jnp.int32),
          pltpu.VMEM((NBUF, C, D), jnp.float32),
          pltpu.SemaphoreType.DMA((NBUF,)),
          pltpu.SemaphoreType.DMA((NBUF,)),
          pltpu.SemaphoreType.DMA((NBUF,)),
      ],
  )
  def sc_embed(x_hbm, tab_hbm, out_hbm, idx_v, rows_v, sem_i, sem_g, sem_o):
    wid = lax.axis_index("s") * NC + lax.axis_index("c")
    base = wid * PER_W

    def idx_start(c, b):
      pltpu.make_async_copy(
          x_hbm.at[pl.ds(base + c * C, C)], idx_v.at[b], sem_i.at[b]).start()

    def idx_wait(b):
      pltpu.make_async_copy(
          x_hbm.at[pl.ds(base, C)], idx_v.at[b], sem_i.at[b]).wait()

    def gather_start(b):
      pltpu.make_async_copy(
          tab_hbm.at[idx_v.at[b]], rows_v.at[b], sem_g.at[b]).start()

    def gather_wait(b):
      pltpu.make_async_copy(
          tab_hbm.at[idx_v.at[b]], rows_v.at[b], sem_g.at[b]).wait()

    def out_start(c, b):
      pltpu.make_async_copy(
          rows_v.at[b], out_hbm.at[pl.ds(base + c * C, C)], sem_o.at[b]).start()

    def out_wait(b):
      pltpu.make_async_copy(
          rows_v.at[b], out_hbm.at[pl.ds(base, C)], sem_o.at[b]).wait()

    def scale(b):
      def body(r, carry):
        for j in range(D // 16):
          sl = pl.ds(j * 16, 16)
          rows_v[b, r, sl] = rows_v[b, r, sl] * SCALE
        return carry
      lax.fori_loop(0, C, body, 0, unroll=4)

    def chunk_body(c, b, *, may_skip_out_wait, gather_next, idx_next):
      b1 = (b + 1) % NBUF
      if gather_next:
        idx_wait(b1)
        if not may_skip_out_wait or b == NBUF - 1:
          out_wait(b1)
        gather_start(b1)
      gather_wait(b)
      scale(b)
      out_start(c, b)
      if idx_next:
        idx_start(c + NBUF, b)

    for b in range(NBUF):
      idx_start(b, b)
    idx_wait(0)
    gather_start(0)

    for b in range(NBUF):
      chunk_body(b, b, may_skip_out_wait=True, gather_next=True, idx_next=True)

    def group(gi, carry):
      g = (gi + 1) * NBUF
      for b in range(NBUF):
        chunk_body(g + b, b, may_skip_out_wait=False,
                   gather_next=True, idx_next=True)
      return carry
    lax.fori_loop(0, NGROUP - 2, group, 0)

    for b in range(NBUF):
      chunk_body(NCHUNK - NBUF + b, b, may_skip_out_wait=False,
                 gather_next=(b != NBUF - 1), idx_next=False)

    for b in range(NBUF):
      out_wait(b)

  return sc_embed


_sc_embed = _build_sc_embed()


def kernel(x, table):
  x_flat = x.reshape(-1).astype(jnp.int32)
  out = _sc_embed(x_flat, table)
  return out.reshape(x.shape + (D,))
